# baseline (speedup 1.0000x reference)
import functools

import jax
import jax.numpy as jnp
from jax import lax
from jax.experimental import pallas as pl
from jax.experimental.pallas import tpu as pltpu
from jax.experimental.pallas import tpu_sc as plsc

VOCAB = 1000000
EMB = 64
B = 16384
L = 50

NC = 2
NS = 16
NW = NC * NS

BPW = B // NW
CHUNK = 128
NBB = BPW // CHUNK
NCHUNK = L * NBB
NBUF = 2
LANES = 16


def _embed_lookup(xt, tab2):
  mesh = plsc.VectorSubcoreMesh(core_axis_name="c", subcore_axis_name="s")

  @functools.partial(
      pl.kernel,
      out_type=jax.ShapeDtypeStruct((L, EMB, B), jnp.float32),
      mesh=mesh,
      scratch_types=(
          [pltpu.VMEM((NCHUNK, CHUNK), jnp.int32),
           pltpu.VMEM((NBUF, CHUNK), jnp.int32),
           pltpu.VMEM((NBUF, CHUNK), jnp.int32)]
          + [pltpu.VMEM((CHUNK, 2 * EMB), jnp.float32) for _ in range(NBUF)]
          + [pltpu.VMEM((EMB, CHUNK), jnp.float32) for _ in range(NBUF)]
          + [pltpu.SemaphoreType.DMA for _ in range(2 * NBUF)]
      ),
      compiler_params=pltpu.CompilerParams(
          use_tc_tiling_on_sc=True, needs_layout_passes=False,
          disable_bounds_checks=True),
  )
  def k(xt_hbm, tab2_hbm, out_hbm, xt_v, q_v, p_v, *rest):
    stag = rest[:NBUF]
    obuf = rest[NBUF:2 * NBUF]
    gsems = rest[2 * NBUF:3 * NBUF]
    osems = rest[3 * NBUF:]

    wid = lax.axis_index("s") * NC + lax.axis_index("c")
    b0w = wid * BPW

    for bb in range(NBB):
      pltpu.sync_copy(xt_hbm.at[:, pl.ds(b0w + bb * CHUNK, CHUNK)],
                      xt_v.at[pl.ds(bb * L, L)])

    def prep_idx(c, b):
      for g in range(CHUNK // LANES):
        v = xt_v[c, pl.ds(g * LANES, LANES)]
        q_v[b, pl.ds(g * LANES, LANES)] = lax.shift_right_logical(v, 1)
        p_v[b, pl.ds(g * LANES, LANES)] = lax.mul(
            lax.bitwise_and(v, 1), jnp.int32(EMB))

    def start_gather(b):
      pltpu.async_copy(tab2_hbm.at[q_v.at[b]], stag[b], gsems[b])

    def wait_gather(b):
      pltpu.make_async_copy(tab2_hbm.at[q_v.at[b]], stag[b], gsems[b]).wait()

    def start_out(c, b):
      l = c % L
      col = b0w + (c // L) * CHUNK
      pltpu.async_copy(obuf[b], out_hbm.at[l, :, pl.ds(col, CHUNK)], osems[b])

    def wait_out(b):
      pltpu.make_async_copy(
          obuf[b], out_hbm.at[0, :, pl.ds(0, CHUNK)], osems[b]).wait()

    def transpose(b):
      for g in range(CHUNK // LANES):
        rvec = jnp.int32(g * LANES) + lax.iota(jnp.int32, LANES)
        pvec = p_v[b, pl.ds(g * LANES, LANES)]
        for e in range(EMB):
          val = plsc.load_gather(stag[b], [rvec, pvec + e])
          obuf[b][e, pl.ds(g * LANES, LANES)] = val

    for b in range(NBUF):
      prep_idx(b, b)
      start_gather(b)

    @pl.loop(0, NCHUNK, step=NBUF)
    def _round(j0):
      for b in range(NBUF):
        c = j0 + b
        wait_gather(b)

        @pl.when(j0 > 0)
        def _():
          wait_out(b)

        transpose(b)
        start_out(c, b)
        cn = c + NBUF

        @pl.when(cn < NCHUNK)
        def _():
          prep_idx(cn, b)
          start_gather(b)

    for b in range(NBUF):
      wait_out(b)

  return k(xt, tab2)


@jax.jit
def kernel(x, table):
  xt = jnp.swapaxes(x.astype(jnp.int32), 0, 1)
  tab2 = jnp.reshape(table, (VOCAB // 2, 2 * EMB))
  out3 = _embed_lookup(xt, tab2)
  return jnp.transpose(out3, (2, 0, 1))


# speedup vs baseline: 1.4803x; 1.4803x over previous
import functools

import jax
import jax.numpy as jnp
from jax import lax
from jax.experimental import pallas as pl
from jax.experimental.pallas import tpu as pltpu
from jax.experimental.pallas import tpu_sc as plsc

VOCAB = 1000000
EMB = 64
B = 16384
L = 50

NC = 2
NS = 16
NW = NC * NS

BPW = B // NW
CHUNK = 128
NBB = BPW // CHUNK
NCHUNK = L * NBB
NBUF = 2
LANES = 16


def _embed_lookup(xt, tab2):
  mesh = plsc.VectorSubcoreMesh(core_axis_name="c", subcore_axis_name="s")

  @functools.partial(
      pl.kernel,
      out_type=jax.ShapeDtypeStruct((L, EMB, B), jnp.float32),
      mesh=mesh,
      scratch_types=(
          [pltpu.VMEM((NCHUNK, CHUNK), jnp.int32),
           pltpu.VMEM((NBUF, CHUNK), jnp.int32),
           pltpu.VMEM((NBUF, CHUNK), jnp.int32)]
          + [pltpu.VMEM((CHUNK, 2 * EMB), jnp.float32) for _ in range(NBUF)]
          + [pltpu.VMEM((EMB, CHUNK), jnp.float32) for _ in range(NBUF)]
          + [pltpu.SemaphoreType.DMA for _ in range(2 * NBUF)]
      ),
      compiler_params=pltpu.CompilerParams(
          use_tc_tiling_on_sc=True, needs_layout_passes=False,
          disable_bounds_checks=True),
  )
  def k(xt_hbm, tab2_hbm, out_hbm, xt_v, q_v, p_v, *rest):
    stag = rest[:NBUF]
    obuf = rest[NBUF:2 * NBUF]
    gsems = rest[2 * NBUF:3 * NBUF]
    osems = rest[3 * NBUF:]

    wid = lax.axis_index("s") * NC + lax.axis_index("c")
    b0w = wid * BPW

    for bb in range(NBB):
      pltpu.sync_copy(xt_hbm.at[:, pl.ds(b0w + bb * CHUNK, CHUNK)],
                      xt_v.at[pl.ds(bb * L, L)])

    def prep_idx(c, b):
      for g in range(CHUNK // LANES):
        v = xt_v[c, pl.ds(g * LANES, LANES)]
        q_v[b, pl.ds(g * LANES, LANES)] = lax.shift_right_logical(v, 1)
        p_v[b, pl.ds(g * LANES, LANES)] = lax.mul(
            lax.bitwise_and(v, 1), jnp.int32(EMB))

    def start_gather(b):
      pltpu.async_copy(tab2_hbm.at[q_v.at[b]], stag[b], gsems[b])

    def wait_gather(b):
      pltpu.make_async_copy(tab2_hbm.at[q_v.at[b]], stag[b], gsems[b]).wait()

    def start_out(c, b):
      l = c % L
      col = b0w + (c // L) * CHUNK
      pltpu.async_copy(obuf[b], out_hbm.at[l, :, pl.ds(col, CHUNK)], osems[b])

    def wait_out(b):
      pltpu.make_async_copy(
          obuf[b], out_hbm.at[0, :, pl.ds(0, CHUNK)], osems[b]).wait()

    lane = lax.iota(jnp.int32, LANES)

    def transpose(b):
      for g in range(CHUNK // LANES):
        rvec = jnp.int32(g * LANES) + lane
        pvec = p_v[b, pl.ds(g * LANES, LANES)]

        @pl.loop(0, EMB, step=LANES)
        def _e(e0):
          for u in range(LANES):
            t = lax.bitwise_and(lane + (e0 + u), jnp.int32(EMB - 1))
            val = plsc.load_gather(stag[b], [rvec, pvec + t])
            plsc.store_scatter(obuf[b], [t, rvec], val)

    for b in range(NBUF):
      prep_idx(b, b)
      start_gather(b)

    @pl.loop(0, NCHUNK, step=NBUF)
    def _round(j0):
      for b in range(NBUF):
        c = j0 + b
        wait_gather(b)

        @pl.when(j0 > 0)
        def _():
          wait_out(b)

        transpose(b)
        start_out(c, b)
        cn = c + NBUF

        @pl.when(cn < NCHUNK)
        def _():
          prep_idx(cn, b)
          start_gather(b)

    for b in range(NBUF):
      wait_out(b)

  return k(xt, tab2)


@jax.jit
def kernel(x, table):
  xt = jnp.swapaxes(x.astype(jnp.int32), 0, 1)
  tab2 = jnp.reshape(table, (VOCAB // 2, 2 * EMB))
  out3 = _embed_lookup(xt, tab2)
  return jnp.transpose(out3, (2, 0, 1))


# speedup vs baseline: 1.8563x; 1.2540x over previous
import functools

import jax
import jax.numpy as jnp
from jax import lax
from jax.experimental import pallas as pl
from jax.experimental.pallas import tpu as pltpu
from jax.experimental.pallas import tpu_sc as plsc

VOCAB = 1000000
EMB = 64
B = 16384
L = 50

NC = 2
NS = 16
NW = NC * NS

BPW = B // NW
CHUNK = 128
NBB = BPW // CHUNK
NCHUNK = L * NBB
NBUF = 4
LANES = 16


def _embed_lookup(xt, tab2):
  mesh = plsc.VectorSubcoreMesh(core_axis_name="c", subcore_axis_name="s")

  @functools.partial(
      pl.kernel,
      out_type=jax.ShapeDtypeStruct((L, EMB, B), jnp.float32),
      mesh=mesh,
      scratch_types=(
          [pltpu.VMEM((NCHUNK, CHUNK), jnp.int32),
           pltpu.VMEM((NBUF, CHUNK), jnp.int32),
           pltpu.VMEM((NBUF, CHUNK), jnp.int32)]
          + [pltpu.VMEM((CHUNK, 2 * EMB), jnp.float32) for _ in range(NBUF)]
          + [pltpu.VMEM((EMB, CHUNK), jnp.float32) for _ in range(NBUF)]
          + [pltpu.SemaphoreType.DMA for _ in range(2 * NBUF)]
      ),
      compiler_params=pltpu.CompilerParams(
          use_tc_tiling_on_sc=True, needs_layout_passes=False,
          disable_bounds_checks=True),
  )
  def k(xt_hbm, tab2_hbm, out_hbm, xt_v, q_v, p_v, *rest):
    stag = rest[:NBUF]
    obuf = rest[NBUF:2 * NBUF]
    gsems = rest[2 * NBUF:3 * NBUF]
    osems = rest[3 * NBUF:]

    wid = lax.axis_index("s") * NC + lax.axis_index("c")
    b0w = wid * BPW

    for bb in range(NBB):
      pltpu.sync_copy(xt_hbm.at[:, pl.ds(b0w + bb * CHUNK, CHUNK)],
                      xt_v.at[pl.ds(bb * L, L)])

    def prep_idx(c, b):
      for g in range(CHUNK // LANES):
        v = xt_v[c, pl.ds(g * LANES, LANES)]
        q_v[b, pl.ds(g * LANES, LANES)] = lax.shift_right_logical(v, 1)
        p_v[b, pl.ds(g * LANES, LANES)] = lax.mul(
            lax.bitwise_and(v, 1), jnp.int32(EMB))

    def start_gather(b):
      pltpu.async_copy(tab2_hbm.at[q_v.at[b]], stag[b], gsems[b])

    def wait_gather(b):
      pltpu.make_async_copy(tab2_hbm.at[q_v.at[b]], stag[b], gsems[b]).wait()

    def start_out(c, b):
      l = c % L
      col = b0w + (c // L) * CHUNK
      pltpu.async_copy(obuf[b], out_hbm.at[l, :, pl.ds(col, CHUNK)], osems[b])

    def wait_out(b):
      pltpu.make_async_copy(
          obuf[b], out_hbm.at[0, :, pl.ds(0, CHUNK)], osems[b]).wait()

    lane = lax.iota(jnp.int32, LANES)

    def transpose(b):
      for g in range(0, CHUNK // LANES, 2):
        rvec0 = jnp.int32(g * LANES) + lane
        rvec1 = jnp.int32((g + 1) * LANES) + lane
        pvec0 = p_v[b, pl.ds(g * LANES, LANES)]
        pvec1 = p_v[b, pl.ds((g + 1) * LANES, LANES)]

        @pl.loop(0, EMB, step=LANES)
        def _e(e0):
          for u in range(LANES):
            t = lax.bitwise_and(lane + (e0 + u), jnp.int32(EMB - 1))
            val0 = plsc.load_gather(stag[b], [rvec0, pvec0 + t])
            val1 = plsc.load_gather(stag[b], [rvec1, pvec1 + t])
            plsc.store_scatter(obuf[b], [t, rvec0], val0)
            plsc.store_scatter(obuf[b], [t, rvec1], val1)

    for b in range(NBUF):
      prep_idx(b, b)
      start_gather(b)

    @pl.loop(0, NCHUNK, step=NBUF)
    def _round(j0):
      for b in range(NBUF):
        c = j0 + b
        wait_gather(b)

        @pl.when(j0 > 0)
        def _():
          wait_out(b)

        transpose(b)
        start_out(c, b)
        cn = c + NBUF

        @pl.when(cn < NCHUNK)
        def _():
          prep_idx(cn, b)
          start_gather(b)

    for b in range(NBUF):
      wait_out(b)

  return k(xt, tab2)


@jax.jit
def kernel(x, table):
  xt = jnp.swapaxes(x.astype(jnp.int32), 0, 1)
  tab2 = jnp.reshape(table, (VOCAB // 2, 2 * EMB))
  out3 = _embed_lookup(xt, tab2)
  return jnp.transpose(out3, (2, 0, 1))


# speedup vs baseline: 1.9917x; 1.0729x over previous
import functools

import jax
import jax.numpy as jnp
from jax import lax
from jax.experimental import pallas as pl
from jax.experimental.pallas import tpu as pltpu
from jax.experimental.pallas import tpu_sc as plsc

VOCAB = 1000000
EMB = 64
B = 16384
L = 50

NC = 2
NS = 16
NW = NC * NS

BPW = B // NW
CHUNK = 128
NBB = BPW // CHUNK
NCHUNK = L * NBB
NBUF = 4
LANES = 16


def _embed_lookup(xt, tab2):
  mesh = plsc.VectorSubcoreMesh(core_axis_name="c", subcore_axis_name="s")

  @functools.partial(
      pl.kernel,
      out_type=jax.ShapeDtypeStruct((L, EMB, B), jnp.float32),
      mesh=mesh,
      scratch_types=(
          [pltpu.VMEM((NCHUNK, CHUNK), jnp.int32)]
          + [pltpu.VMEM((CHUNK, 2 * EMB), jnp.float32) for _ in range(NBUF)]
          + [pltpu.VMEM((EMB, CHUNK), jnp.float32) for _ in range(NBUF)]
          + [pltpu.SemaphoreType.DMA for _ in range(2 * NBUF)]
      ),
      compiler_params=pltpu.CompilerParams(
          use_tc_tiling_on_sc=True, needs_layout_passes=False,
          disable_bounds_checks=True),
  )
  def k(xt_hbm, tab2_hbm, out_hbm, xt_v, *rest):
    stag = rest[:NBUF]
    obuf = rest[NBUF:2 * NBUF]
    gsems = rest[2 * NBUF:3 * NBUF]
    osems = rest[3 * NBUF:]

    wid = lax.axis_index("s") * NC + lax.axis_index("c")
    b0w = wid * BPW

    for bb in range(NBB):
      pltpu.sync_copy(xt_hbm.at[:, pl.ds(b0w + bb * CHUNK, CHUNK)],
                      xt_v.at[pl.ds(bb * L, L)])

    def start_gather(c, b):
      pltpu.async_copy(tab2_hbm.at[xt_v.at[c]], stag[b], gsems[b])

    def wait_gather(b):
      pltpu.make_async_copy(tab2_hbm.at[xt_v.at[0]], stag[b], gsems[b]).wait()

    def start_out(c, b):
      l = c % L
      col = b0w + (c // L) * CHUNK
      pltpu.async_copy(obuf[b], out_hbm.at[l, :, pl.ds(col, CHUNK)], osems[b])

    def wait_out(b):
      pltpu.make_async_copy(
          obuf[b], out_hbm.at[0, :, pl.ds(0, CHUNK)], osems[b]).wait()

    lane = lax.iota(jnp.int32, LANES)

    def transpose(b):
      for g in range(0, CHUNK // LANES, 2):
        rvec0 = jnp.int32(g * LANES) + lane
        rvec1 = jnp.int32((g + 1) * LANES) + lane

        @pl.loop(0, EMB, step=LANES)
        def _e(e0):
          for u in range(LANES):
            t = lax.bitwise_and(lane + (e0 + u), jnp.int32(EMB - 1))
            val0 = plsc.load_gather(stag[b], [rvec0, t])
            val1 = plsc.load_gather(stag[b], [rvec1, t])
            plsc.store_scatter(obuf[b], [t, rvec0], val0)
            plsc.store_scatter(obuf[b], [t, rvec1], val1)

    for b in range(NBUF):
      start_gather(b, b)

    @pl.loop(0, NCHUNK, step=NBUF)
    def _round(j0):
      for b in range(NBUF):
        c = j0 + b
        wait_gather(b)

        @pl.when(j0 > 0)
        def _():
          wait_out(b)

        transpose(b)
        start_out(c, b)
        cn = c + NBUF

        @pl.when(cn < NCHUNK)
        def _():
          start_gather(cn, b)

    for b in range(NBUF):
      wait_out(b)

  return k(xt, tab2)


@jax.jit
def kernel(x, table):
  xt = jnp.swapaxes(x.astype(jnp.int32), 0, 1)
  tab2 = jnp.pad(table, ((0, 0), (0, EMB)))
  out3 = _embed_lookup(xt, tab2)
  return jnp.transpose(out3, (2, 0, 1))
